# baseline (speedup 1.0000x reference)
import functools

import jax
import jax.numpy as jnp
from jax import lax
from jax.experimental import pallas as pl
from jax.experimental.pallas import tpu as pltpu

N_MIRNA = 4000
N_GENE = 6000
N = N_MIRNA + N_GENE
R = 8
NB_DIM = 10
E = 160000
IN_CH = 256
HID_CH = 256
OUT_CH = 128

BN = 1000
NBLK = N // BN


def _proj_body(xm_ref, xg_ref, wm_ref, wg_ref, bm_ref, bg_ref, out_ref):
    i = pl.program_id(0)
    nm_blocks = N_MIRNA // BN

    @pl.when(i < nm_blocks)
    def _():
        res = lax.dot_general(xm_ref[...], wm_ref[...],
                              (((1,), (1,)), ((), ())),
                              preferred_element_type=jnp.float32)
        out_ref[...] = res + bm_ref[0]

    @pl.when(i >= nm_blocks)
    def _():
        res = lax.dot_general(xg_ref[...], wg_ref[...],
                              (((1,), (1,)), ((), ())),
                              preferred_element_type=jnp.float32)
        out_ref[...] = res + bg_ref[0]


def _project(x_mirna, x_gene, wm, wg, bm, bg):
    nm_blocks = N_MIRNA // BN
    return pl.pallas_call(
        _proj_body,
        grid=(NBLK,),
        in_specs=[
            pl.BlockSpec((BN, x_mirna.shape[1]),
                         lambda i: (jnp.minimum(i, nm_blocks - 1), 0)),
            pl.BlockSpec((BN, x_gene.shape[1]),
                         lambda i: (jnp.maximum(i - nm_blocks, 0), 0)),
            pl.BlockSpec(wm.shape, lambda i: (0, 0)),
            pl.BlockSpec(wg.shape, lambda i: (0, 0)),
            pl.BlockSpec((1, IN_CH), lambda i: (0, 0)),
            pl.BlockSpec((1, IN_CH), lambda i: (0, 0)),
        ],
        out_specs=pl.BlockSpec((BN, IN_CH), lambda i: (i, 0)),
        out_shape=jax.ShapeDtypeStruct((N, IN_CH), jnp.float32),
    )(x_mirna, x_gene, wm, wg, bm.reshape(1, -1), bg.reshape(1, -1))


def _xw_body(x_ref, comp_ref, bases_ref, out_ref, w_ref, *, in_ch, o_ch):
    nb = pl.program_id(0)
    r = pl.program_id(1)

    @pl.when((nb == 0) & (r == 0))
    def _():
        bflat = bases_ref[...].reshape(NB_DIM, in_ch * o_ch)
        wall = lax.dot_general(comp_ref[...], bflat,
                               (((1,), (0,)), ((), ())),
                               preferred_element_type=jnp.float32)
        w_ref[...] = wall.reshape(R, in_ch, o_ch)

    wr = w_ref[r]
    res = lax.dot_general(x_ref[...], wr, (((1,), (0,)), ((), ())),
                          preferred_element_type=jnp.float32)
    out_ref[0] = res.reshape(BN, 2, o_ch // 2)


def _xw(x, comp, bases, in_ch, o_ch):
    return pl.pallas_call(
        functools.partial(_xw_body, in_ch=in_ch, o_ch=o_ch),
        grid=(NBLK, R),
        in_specs=[
            pl.BlockSpec((BN, in_ch), lambda nb, r: (nb, 0)),
            pl.BlockSpec((R, NB_DIM), lambda nb, r: (0, 0)),
            pl.BlockSpec((NB_DIM, in_ch, o_ch), lambda nb, r: (0, 0, 0)),
        ],
        out_specs=pl.BlockSpec((1, BN, 2, o_ch // 2),
                               lambda nb, r: (r, nb, 0, 0)),
        out_shape=jax.ShapeDtypeStruct((R, N, 2, o_ch // 2), jnp.float32),
        scratch_shapes=[pltpu.VMEM((R, in_ch, o_ch), jnp.float32)],
    )(x, comp, bases)


def _combine_body(agg_ref, x_ref, root_ref, bias_ref, out_ref, *, o_ch):
    oh = o_ch // 2
    s = lax.dot_general(x_ref[...], root_ref[...], (((1,), (0,)), ((), ())),
                        preferred_element_type=jnp.float32)
    s = s + bias_ref[0]
    out_ref[:, :oh] = agg_ref[0] + s[:, :oh]
    out_ref[:, oh:] = agg_ref[1] + s[:, oh:]


def _combine(agg, x, root, bias, in_ch, o_ch):
    return pl.pallas_call(
        functools.partial(_combine_body, o_ch=o_ch),
        grid=(NBLK,),
        in_specs=[
            pl.BlockSpec((2, BN, o_ch // 2), lambda i: (0, i, 0)),
            pl.BlockSpec((BN, in_ch), lambda i: (i, 0)),
            pl.BlockSpec((in_ch, o_ch), lambda i: (0, 0)),
            pl.BlockSpec((1, o_ch), lambda i: (0, 0)),
        ],
        out_specs=pl.BlockSpec((BN, o_ch), lambda i: (i, 0)),
        out_shape=jax.ShapeDtypeStruct((N, o_ch), jnp.float32),
    )(agg, x, root, bias.reshape(1, -1))


def _aggregate(xw, src, dst, rel, w, o_ch):
    oh = o_ch // 2
    xwf = xw.reshape(R, N, o_ch)
    msg = xwf[rel, src] * w[:, None]
    agg = jax.ops.segment_sum(msg, dst, num_segments=N)
    return jnp.stack([agg[:, :oh], agg[:, oh:]], axis=0)


def kernel(x_mirna, x_gene, edge_index, edge_type, lin_m_w, lin_m_b,
           lin_g_w, lin_g_b, comp1, bases1, root1, bias1, comp2, bases2,
           root2, bias2):
    src = edge_index[0].astype(jnp.int32)
    dst = edge_index[1].astype(jnp.int32)
    rel = edge_type.astype(jnp.int32)

    x = _project(x_mirna, x_gene, lin_m_w, lin_g_w, lin_m_b, lin_g_b)

    key = dst * R + rel
    cnt = jax.ops.segment_sum(jnp.ones((E,), jnp.float32), key,
                              num_segments=N * R)
    w = 1.0 / jnp.maximum(cnt, 1.0)[key]

    xw1 = _xw(x, comp1, bases1, IN_CH, HID_CH)
    agg1 = _aggregate(xw1, src, dst, rel, w, HID_CH)
    h = _combine(agg1, x, root1, bias1, IN_CH, HID_CH)

    xw2 = _xw(h, comp2, bases2, HID_CH, OUT_CH)
    agg2 = _aggregate(xw2, src, dst, rel, w, OUT_CH)
    out = _combine(agg2, h, root2, bias2, HID_CH, OUT_CH)

    return (out[:N_MIRNA], out[N_MIRNA:])


# speedup vs baseline: 8.1921x; 8.1921x over previous
import functools

import jax
import jax.numpy as jnp
from jax import lax
from jax.experimental import pallas as pl
from jax.experimental.pallas import tpu as pltpu
from jax.experimental.pallas import tpu_sc as plsc

N_MIRNA = 4000
N_GENE = 6000
N = N_MIRNA + N_GENE
R = 8
NB_DIM = 10
E = 160000
IN_CH = 256
HID_CH = 256
OUT_CH = 128

BN = 1000
NBLK = N // BN


def _f1_body(xm_ref, xg_ref, wm_ref, wg_ref, bm_ref, bg_ref, comp_ref,
             bases_ref, xw_ref, x_ref, w_ref, xs_ref):
    nb = pl.program_id(0)
    r = pl.program_id(1)
    nm_blocks = N_MIRNA // BN

    @pl.when((nb == 0) & (r == 0))
    def _():
        bflat = bases_ref[...].reshape(NB_DIM, IN_CH * HID_CH)
        wall = lax.dot_general(comp_ref[...], bflat,
                               (((1,), (0,)), ((), ())),
                               preferred_element_type=jnp.float32)
        w_ref[...] = wall.reshape(R, IN_CH, HID_CH)

    @pl.when((r == 0) & (nb < nm_blocks))
    def _():
        res = lax.dot_general(xm_ref[...], wm_ref[...],
                              (((1,), (1,)), ((), ())),
                              preferred_element_type=jnp.float32)
        xs_ref[...] = res + bm_ref[0]
        x_ref[...] = xs_ref[...]

    @pl.when((r == 0) & (nb >= nm_blocks))
    def _():
        res = lax.dot_general(xg_ref[...], wg_ref[...],
                              (((1,), (1,)), ((), ())),
                              preferred_element_type=jnp.float32)
        xs_ref[...] = res + bg_ref[0]
        x_ref[...] = xs_ref[...]

    res = lax.dot_general(xs_ref[...], w_ref[r], (((1,), (0,)), ((), ())),
                          preferred_element_type=jnp.float32)
    xw_ref[0] = res


def _proj_xw1(x_mirna, x_gene, wm, wg, bm, bg, comp, bases):
    nm_blocks = N_MIRNA // BN
    return pl.pallas_call(
        _f1_body,
        grid=(NBLK, R),
        in_specs=[
            pl.BlockSpec((BN, x_mirna.shape[1]),
                         lambda nb, r: (jnp.minimum(nb, nm_blocks - 1), 0)),
            pl.BlockSpec((BN, x_gene.shape[1]),
                         lambda nb, r: (jnp.maximum(nb - nm_blocks, 0), 0)),
            pl.BlockSpec(wm.shape, lambda nb, r: (0, 0)),
            pl.BlockSpec(wg.shape, lambda nb, r: (0, 0)),
            pl.BlockSpec((1, IN_CH), lambda nb, r: (0, 0)),
            pl.BlockSpec((1, IN_CH), lambda nb, r: (0, 0)),
            pl.BlockSpec((R, NB_DIM), lambda nb, r: (0, 0)),
            pl.BlockSpec((NB_DIM, IN_CH, HID_CH), lambda nb, r: (0, 0, 0)),
        ],
        out_specs=[
            pl.BlockSpec((1, BN, HID_CH), lambda nb, r: (r, nb, 0)),
            pl.BlockSpec((BN, IN_CH), lambda nb, r: (nb, 0)),
        ],
        out_shape=[jax.ShapeDtypeStruct((R, N, HID_CH), jnp.float32),
                   jax.ShapeDtypeStruct((N, IN_CH), jnp.float32)],
        scratch_shapes=[pltpu.VMEM((R, IN_CH, HID_CH), jnp.float32),
                        pltpu.VMEM((BN, IN_CH), jnp.float32)],
    )(x_mirna, x_gene, wm, wg, bm.reshape(1, -1), bg.reshape(1, -1),
      comp, bases)


def _f2_body(agg_ref, x_ref, root_ref, bias_ref, comp_ref, bases_ref,
             xw_ref, h_ref, w_ref, hs_ref):
    nb = pl.program_id(0)
    r = pl.program_id(1)

    @pl.when((nb == 0) & (r == 0))
    def _():
        bflat = bases_ref[...].reshape(NB_DIM, HID_CH * OUT_CH)
        wall = lax.dot_general(comp_ref[...], bflat,
                               (((1,), (0,)), ((), ())),
                               preferred_element_type=jnp.float32)
        w_ref[...] = wall.reshape(R, HID_CH, OUT_CH)

    @pl.when(r == 0)
    def _():
        oh = HID_CH // 2
        sv = lax.dot_general(x_ref[...], root_ref[...],
                             (((1,), (0,)), ((), ())),
                             preferred_element_type=jnp.float32)
        sv = sv + bias_ref[0]
        hs_ref[:, :oh] = agg_ref[0] + sv[:, :oh]
        hs_ref[:, oh:] = agg_ref[1] + sv[:, oh:]
        h_ref[...] = hs_ref[...]

    res = lax.dot_general(hs_ref[...], w_ref[r], (((1,), (0,)), ((), ())),
                          preferred_element_type=jnp.float32)
    xw_ref[0] = res


def _combine_xw2(agg, x, root, bias, comp, bases):
    return pl.pallas_call(
        _f2_body,
        grid=(NBLK, R),
        in_specs=[
            pl.BlockSpec((2, BN, HID_CH // 2), lambda nb, r: (0, nb, 0)),
            pl.BlockSpec((BN, IN_CH), lambda nb, r: (nb, 0)),
            pl.BlockSpec((IN_CH, HID_CH), lambda nb, r: (0, 0)),
            pl.BlockSpec((1, HID_CH), lambda nb, r: (0, 0)),
            pl.BlockSpec((R, NB_DIM), lambda nb, r: (0, 0)),
            pl.BlockSpec((NB_DIM, HID_CH, OUT_CH), lambda nb, r: (0, 0, 0)),
        ],
        out_specs=[
            pl.BlockSpec((1, BN, OUT_CH), lambda nb, r: (r, nb, 0)),
            pl.BlockSpec((BN, HID_CH), lambda nb, r: (nb, 0)),
        ],
        out_shape=[jax.ShapeDtypeStruct((R, N, OUT_CH), jnp.float32),
                   jax.ShapeDtypeStruct((N, HID_CH), jnp.float32)],
        scratch_shapes=[pltpu.VMEM((R, HID_CH, OUT_CH), jnp.float32),
                        pltpu.VMEM((BN, HID_CH), jnp.float32)],
    )(agg, x, root, bias.reshape(1, -1), comp, bases)


def _combine_body(agg_ref, x_ref, root_ref, bias_ref, out_ref, *, o_ch,
                  split):
    s = lax.dot_general(x_ref[...], root_ref[...], (((1,), (0,)), ((), ())),
                        preferred_element_type=jnp.float32)
    s = s + bias_ref[0]
    if split:
        oh = o_ch // 2
        out_ref[:, :oh] = agg_ref[0] + s[:, :oh]
        out_ref[:, oh:] = agg_ref[1] + s[:, oh:]
    else:
        out_ref[...] = agg_ref[0] + agg_ref[1] + s


def _combine(agg, x, root, bias, in_ch, o_ch, split):
    aw = o_ch // 2 if split else o_ch
    return pl.pallas_call(
        functools.partial(_combine_body, o_ch=o_ch, split=split),
        grid=(NBLK,),
        in_specs=[
            pl.BlockSpec((2, BN, aw), lambda i: (0, i, 0)),
            pl.BlockSpec((BN, in_ch), lambda i: (i, 0)),
            pl.BlockSpec((in_ch, o_ch), lambda i: (0, 0)),
            pl.BlockSpec((1, o_ch), lambda i: (0, 0)),
        ],
        out_specs=pl.BlockSpec((BN, o_ch), lambda i: (i, 0)),
        out_shape=jax.ShapeDtypeStruct((N, o_ch), jnp.float32),
    )(agg, x, root, bias.reshape(1, -1))


NC, NS = 2, 16
NW = NC * NS
K = 128
E_PAD = 163840
EB1 = E // NW
EB2 = E_PAD // NW
EM = E_PAD // NS
CM = EM // K
N_ACC = 10240
NSTRIPE = N_ACC // NS

_MESH = plsc.VectorSubcoreMesh(core_axis_name="c", subcore_axis_name="s")


ZSTRIPE = 5120
NRP = NS * ZSTRIPE


@functools.partial(
    pl.kernel,
    out_type=[jax.ShapeDtypeStruct((E_PAD,), jnp.float32),
              jax.ShapeDtypeStruct((E_PAD,), jnp.int32)],
    mesh=_MESH,
    compiler_params=pltpu.CompilerParams(needs_layout_passes=False),
    scratch_types=[
        pltpu.VMEM((EM,), jnp.int32),
        pltpu.VMEM((EM,), jnp.int32),
        pltpu.VMEM((EM,), jnp.int32),
        pltpu.VMEM((EM,), jnp.float32),
        pltpu.VMEM((1, K), jnp.int32),
        pltpu.VMEM((1, K), jnp.int32),
        pltpu.VMEM((K,), jnp.int32),
        pltpu.VMEM((ZSTRIPE,), jnp.int32),
        pltpu.VMEM_SHARED((NRP,), jnp.int32),
    ],
)
def _edge_meta_kernel(src_hbm, dst_hbm, rel_hbm, zeros_hbm, w_hbm, g2_hbm,
                      bufa, relb, g2cb, wcb, keyb, cntc, onesb, stage, csh):
    c = lax.axis_index("c")
    s = lax.axis_index("s")
    base = s * EM
    pltpu.sync_copy(rel_hbm.at[pl.ds(base, EM)], relb)

    @pl.when(c == 1)
    def _():
        pltpu.sync_copy(src_hbm.at[pl.ds(base, EM)], bufa)

        def mk(i, _):
            sl = pl.ds(i * 16, 16)
            g2cb[sl] = (relb[sl] * N + bufa[sl]) * 2
            return 0
        lax.fori_loop(0, EM // 16, mk, 0)
        pltpu.sync_copy(g2cb, g2_hbm.at[pl.ds(base, EM)])

    @pl.when(c == 0)
    def _():
        pltpu.sync_copy(zeros_hbm, stage)
        pltpu.sync_copy(stage, csh.at[pl.ds(s * ZSTRIPE, ZSTRIPE)])
        pltpu.sync_copy(dst_hbm.at[pl.ds(base, EM)], bufa)
        for v in range(K // 16):
            onesb[pl.ds(v * 16, 16)] = jnp.ones((16,), jnp.int32)
        plsc.subcore_barrier()

        def mkkeys(j):
            for v in range(K // 16):
                esl = pl.ds(j * K + v * 16, 16)
                eid = base + j * K + v * 16 + lax.iota(jnp.int32, 16)
                key = jnp.where(eid < E, bufa[esl] * R + relb[esl],
                                N * R + (eid & 127))
                keyb[0, pl.ds(v * 16, 16)] = key

        def hchunk(j, _):
            mkkeys(j)
            pltpu.sync_copy(onesb, csh.at[keyb.at[0]], add=True)
            return 0
        lax.fori_loop(0, CM, hchunk, 0)
        plsc.subcore_barrier()

        def wchunk(j, _):
            mkkeys(j)
            pltpu.sync_copy(csh.at[keyb.at[0]], cntc.at[0])
            for v in range(K // 16):
                sl = pl.ds(v * 16, 16)
                cf = cntc[0, sl].astype(jnp.float32)
                eid = base + j * K + v * 16 + lax.iota(jnp.int32, 16)
                wcb[pl.ds(j * K + v * 16, 16)] = jnp.where(
                    eid < E, 1.0 / jnp.maximum(cf, 1.0), 0.0)
            return 0
        lax.fori_loop(0, CM, wchunk, 0)
        pltpu.sync_copy(wcb, w_hbm.at[pl.ds(base, EM)])


MG = 8


def _agg_body(oh, split, xw_hbm, g2_hbm, dst_hbm, w_hbm, zeros_hbm, agg_hbm,
              g2b, dstb, wb, idxb, rowsb, acc, gs0, gs1, ss0, ss1):
    c = lax.axis_index("c")
    s = lax.axis_index("s")
    pltpu.sync_copy(zeros_hbm, rowsb.at[0])
    for q in range(NSTRIPE // K):
        pltpu.async_copy(rowsb.at[0], acc.at[pl.ds(s * NSTRIPE + q * K, K)],
                         gs0)
    for q in range(NSTRIPE // K):
        pltpu.make_async_copy(rowsb.at[0],
                              acc.at[pl.ds(s * NSTRIPE + q * K, K)],
                              gs0).wait()
    plsc.subcore_barrier()
    ngroups = (CM if split else CM // 2) // MG
    gsems = (gs0, gs1)
    ssems = (ss0, ss1)

    def mkidx(u):
        b = u & 1
        for v in range(K // 16):
            sl = pl.ds(v * 16, 16)
            if split:
                idxb[b, sl] = g2b[u, sl] + c
            else:
                idxb[b, sl] = g2b[u, sl] >> 1

    def group(t, _):
        grow = (s * CM + t * MG) if split else (s * CM + c * (CM // 2) + t * MG)
        pltpu.sync_copy(g2_hbm.at[pl.ds(grow, MG)], g2b)
        pltpu.sync_copy(dst_hbm.at[pl.ds(grow, MG)], dstb)
        pltpu.sync_copy(w_hbm.at[pl.ds(grow, MG)], wb)
        mkidx(0)
        pltpu.async_copy(xw_hbm.at[idxb.at[0]], rowsb.at[0], gsems[0])
        for u in range(MG):
            b = u & 1
            b1 = (u + 1) & 1
            if u + 1 < MG:
                mkidx(u + 1)
                if u >= 1:
                    pltpu.make_async_copy(rowsb.at[b1],
                                          acc.at[dstb.at[u - 1]],
                                          ssems[b1]).wait()
                pltpu.async_copy(xw_hbm.at[idxb.at[b1]], rowsb.at[b1],
                                 gsems[b1])
            pltpu.make_async_copy(xw_hbm.at[idxb.at[b]], rowsb.at[b],
                                  gsems[b]).wait()

            def scale(g, _):
                wv = wb[u, pl.ds(g * 16, 16)]
                for l in range(16):
                    wk = wv[l]
                    k = g * 16 + l
                    for f in range(oh // 16):
                        fl = pl.ds(f * 16, 16)
                        rowsb[b, k, fl] = rowsb[b, k, fl] * wk
                return 0
            lax.fori_loop(0, K // 16, scale, 0)
            pltpu.async_copy(rowsb.at[b], acc.at[dstb.at[u]], ssems[b],
                             add=True)
        pltpu.make_async_copy(rowsb.at[0], acc.at[dstb.at[MG - 2]],
                              ssems[0]).wait()
        pltpu.make_async_copy(rowsb.at[1], acc.at[dstb.at[MG - 1]],
                              ssems[1]).wait()
        return 0
    lax.fori_loop(0, ngroups, group, 0)
    plsc.subcore_barrier()
    for q in range(NSTRIPE // K):
        b = q & 1
        rsl = pl.ds(s * NSTRIPE + q * K, K)
        if q >= 2:
            psl = pl.ds(s * NSTRIPE + (q - 2) * K, K)
            pltpu.make_async_copy(rowsb.at[b], agg_hbm.at[c, psl],
                                  ssems[b]).wait()
        pltpu.sync_copy(acc.at[rsl], rowsb.at[b])
        pltpu.async_copy(rowsb.at[b], agg_hbm.at[c, rsl], ssems[b])
    for q in range(NSTRIPE // K - 2, NSTRIPE // K):
        b = q & 1
        rsl = pl.ds(s * NSTRIPE + q * K, K)
        pltpu.make_async_copy(rowsb.at[b], agg_hbm.at[c, rsl],
                              ssems[b]).wait()


def _aggregate_sc(xw, g2_2d, dst_2d, w_2d, zeros, o_ch, split):
    oh = o_ch // 2 if split else o_ch
    xwf = xw.reshape(-1, oh)
    return pl.kernel(
        functools.partial(_agg_body, oh, split),
        out_type=jax.ShapeDtypeStruct((NC, N_ACC, oh), jnp.float32),
        mesh=_MESH,
        compiler_params=pltpu.CompilerParams(needs_layout_passes=False),
        scratch_types=[
            pltpu.VMEM((MG, K), jnp.int32),
            pltpu.VMEM((MG, K), jnp.int32),
            pltpu.VMEM((MG, K), jnp.float32),
            pltpu.VMEM((2, K), jnp.int32),
            pltpu.VMEM((2, K, oh), jnp.float32),
            pltpu.VMEM_SHARED((N_ACC, oh), jnp.float32),
            pltpu.SemaphoreType.DMA,
            pltpu.SemaphoreType.DMA,
            pltpu.SemaphoreType.DMA,
            pltpu.SemaphoreType.DMA,
        ],
    )(xwf, g2_2d, dst_2d, w_2d, zeros)


def kernel(x_mirna, x_gene, edge_index, edge_type, lin_m_w, lin_m_b,
           lin_g_w, lin_g_b, comp1, bases1, root1, bias1, comp2, bases2,
           root2, bias2):
    src = edge_index[0].astype(jnp.int32)
    dst = edge_index[1].astype(jnp.int32)
    rel = edge_type.astype(jnp.int32)

    npad = E_PAD - E
    spread = jnp.arange(npad, dtype=jnp.int32) % 997
    src_p = jnp.concatenate([src, spread])
    dst_p = jnp.concatenate([dst, spread])
    rel_p = jnp.concatenate([rel, jnp.zeros((npad,), jnp.int32)])

    zeros_h = jnp.zeros((ZSTRIPE,), jnp.int32)
    w, g2 = _edge_meta_kernel(src_p, dst_p, rel_p, zeros_h)
    w_2d = w.reshape(E_PAD // K, K)
    g2_2d = g2.reshape(E_PAD // K, K)
    dst_2d = dst_p.reshape(E_PAD // K, K)
    zeros1 = jnp.zeros((K, HID_CH // 2), jnp.float32)
    zeros2 = jnp.zeros((K, OUT_CH), jnp.float32)

    xw1, x = _proj_xw1(x_mirna, x_gene, lin_m_w, lin_g_w, lin_m_b, lin_g_b,
                       comp1, bases1)
    agg1 = _aggregate_sc(xw1, g2_2d, dst_2d, w_2d, zeros1, HID_CH, True)

    xw2, h = _combine_xw2(agg1, x, root1, bias1, comp2, bases2)
    agg2 = _aggregate_sc(xw2, g2_2d, dst_2d, w_2d, zeros2, OUT_CH, False)
    out = _combine(agg2, h, root2, bias2, HID_CH, OUT_CH, False)

    return (out[:N_MIRNA], out[N_MIRNA:])


# speedup vs baseline: 8.9058x; 1.0871x over previous
import functools

import jax
import jax.numpy as jnp
from jax import lax
from jax.experimental import pallas as pl
from jax.experimental.pallas import tpu as pltpu
from jax.experimental.pallas import tpu_sc as plsc

N_MIRNA = 4000
N_GENE = 6000
N = N_MIRNA + N_GENE
R = 8
NB_DIM = 10
E = 160000
IN_CH = 256
HID_CH = 256
OUT_CH = 128

BN = 2000
NBLK = N // BN


def _f1_body(xm_ref, xg_ref, wm_ref, wg_ref, bm_ref, bg_ref, comp_ref,
             bases_ref, xw_ref, x_ref, w_ref, xs_ref):
    nb = pl.program_id(0)
    r = pl.program_id(1)
    nm_blocks = N_MIRNA // BN

    @pl.when((nb == 0) & (r == 0))
    def _():
        bflat = bases_ref[...].reshape(NB_DIM, IN_CH * HID_CH)
        wall = lax.dot_general(comp_ref[...], bflat,
                               (((1,), (0,)), ((), ())),
                               preferred_element_type=jnp.float32)
        w_ref[...] = wall.reshape(R, IN_CH, HID_CH)

    @pl.when((r == 0) & (nb < nm_blocks))
    def _():
        res = lax.dot_general(xm_ref[...], wm_ref[...],
                              (((1,), (1,)), ((), ())),
                              preferred_element_type=jnp.float32)
        xs_ref[...] = res + bm_ref[0]
        x_ref[...] = xs_ref[...]

    @pl.when((r == 0) & (nb >= nm_blocks))
    def _():
        res = lax.dot_general(xg_ref[...], wg_ref[...],
                              (((1,), (1,)), ((), ())),
                              preferred_element_type=jnp.float32)
        xs_ref[...] = res + bg_ref[0]
        x_ref[...] = xs_ref[...]

    res = lax.dot_general(xs_ref[...], w_ref[r], (((1,), (0,)), ((), ())),
                          preferred_element_type=jnp.float32)
    xw_ref[0] = res


def _proj_xw1(x_mirna, x_gene, wm, wg, bm, bg, comp, bases):
    nm_blocks = N_MIRNA // BN
    return pl.pallas_call(
        _f1_body,
        grid=(NBLK, R),
        in_specs=[
            pl.BlockSpec((BN, x_mirna.shape[1]),
                         lambda nb, r: (jnp.minimum(nb, nm_blocks - 1), 0)),
            pl.BlockSpec((BN, x_gene.shape[1]),
                         lambda nb, r: (jnp.maximum(nb - nm_blocks, 0), 0)),
            pl.BlockSpec(wm.shape, lambda nb, r: (0, 0)),
            pl.BlockSpec(wg.shape, lambda nb, r: (0, 0)),
            pl.BlockSpec((1, IN_CH), lambda nb, r: (0, 0)),
            pl.BlockSpec((1, IN_CH), lambda nb, r: (0, 0)),
            pl.BlockSpec((R, NB_DIM), lambda nb, r: (0, 0)),
            pl.BlockSpec((NB_DIM, IN_CH, HID_CH), lambda nb, r: (0, 0, 0)),
        ],
        out_specs=[
            pl.BlockSpec((1, BN, HID_CH), lambda nb, r: (r, nb, 0)),
            pl.BlockSpec((BN, IN_CH), lambda nb, r: (nb, 0)),
        ],
        out_shape=[jax.ShapeDtypeStruct((R, N, HID_CH), jnp.float32),
                   jax.ShapeDtypeStruct((N, IN_CH), jnp.float32)],
        scratch_shapes=[pltpu.VMEM((R, IN_CH, HID_CH), jnp.float32),
                        pltpu.VMEM((BN, IN_CH), jnp.float32)],
    )(x_mirna, x_gene, wm, wg, bm.reshape(1, -1), bg.reshape(1, -1),
      comp, bases)


def _f2_body(agg_ref, x_ref, root_ref, bias_ref, comp_ref, bases_ref,
             xw_ref, h_ref, w_ref, hs_ref):
    nb = pl.program_id(0)
    r = pl.program_id(1)

    @pl.when((nb == 0) & (r == 0))
    def _():
        bflat = bases_ref[...].reshape(NB_DIM, HID_CH * OUT_CH)
        wall = lax.dot_general(comp_ref[...], bflat,
                               (((1,), (0,)), ((), ())),
                               preferred_element_type=jnp.float32)
        w_ref[...] = wall.reshape(R, HID_CH, OUT_CH)

    @pl.when(r == 0)
    def _():
        oh = HID_CH // 2
        sv = lax.dot_general(x_ref[...], root_ref[...],
                             (((1,), (0,)), ((), ())),
                             preferred_element_type=jnp.float32)
        sv = sv + bias_ref[0]
        hs_ref[:, :oh] = agg_ref[0] + sv[:, :oh]
        hs_ref[:, oh:] = agg_ref[1] + sv[:, oh:]
        h_ref[...] = hs_ref[...]

    res = lax.dot_general(hs_ref[...], w_ref[r], (((1,), (0,)), ((), ())),
                          preferred_element_type=jnp.float32)
    xw_ref[0] = res


def _combine_xw2(agg, x, root, bias, comp, bases):
    return pl.pallas_call(
        _f2_body,
        grid=(NBLK, R),
        in_specs=[
            pl.BlockSpec((2, BN, HID_CH // 2), lambda nb, r: (0, nb, 0)),
            pl.BlockSpec((BN, IN_CH), lambda nb, r: (nb, 0)),
            pl.BlockSpec((IN_CH, HID_CH), lambda nb, r: (0, 0)),
            pl.BlockSpec((1, HID_CH), lambda nb, r: (0, 0)),
            pl.BlockSpec((R, NB_DIM), lambda nb, r: (0, 0)),
            pl.BlockSpec((NB_DIM, HID_CH, OUT_CH), lambda nb, r: (0, 0, 0)),
        ],
        out_specs=[
            pl.BlockSpec((1, BN, OUT_CH), lambda nb, r: (r, nb, 0)),
            pl.BlockSpec((BN, HID_CH), lambda nb, r: (nb, 0)),
        ],
        out_shape=[jax.ShapeDtypeStruct((R, N, OUT_CH), jnp.float32),
                   jax.ShapeDtypeStruct((N, HID_CH), jnp.float32)],
        scratch_shapes=[pltpu.VMEM((R, HID_CH, OUT_CH), jnp.float32),
                        pltpu.VMEM((BN, HID_CH), jnp.float32)],
    )(agg, x, root, bias.reshape(1, -1), comp, bases)


def _combine_body(agg_ref, x_ref, root_ref, bias_ref, out_ref, *, o_ch,
                  split):
    s = lax.dot_general(x_ref[...], root_ref[...], (((1,), (0,)), ((), ())),
                        preferred_element_type=jnp.float32)
    s = s + bias_ref[0]
    if split:
        oh = o_ch // 2
        out_ref[:, :oh] = agg_ref[0] + s[:, :oh]
        out_ref[:, oh:] = agg_ref[1] + s[:, oh:]
    else:
        out_ref[...] = agg_ref[0] + agg_ref[1] + s


def _combine(agg, x, root, bias, in_ch, o_ch, split):
    aw = o_ch // 2 if split else o_ch
    return pl.pallas_call(
        functools.partial(_combine_body, o_ch=o_ch, split=split),
        grid=(NBLK,),
        in_specs=[
            pl.BlockSpec((2, BN, aw), lambda i: (0, i, 0)),
            pl.BlockSpec((BN, in_ch), lambda i: (i, 0)),
            pl.BlockSpec((in_ch, o_ch), lambda i: (0, 0)),
            pl.BlockSpec((1, o_ch), lambda i: (0, 0)),
        ],
        out_specs=pl.BlockSpec((BN, o_ch), lambda i: (i, 0)),
        out_shape=jax.ShapeDtypeStruct((N, o_ch), jnp.float32),
    )(agg, x, root, bias.reshape(1, -1))


NC, NS = 2, 16
NW = NC * NS
K = 128
E_PAD = 163840
EB1 = E // NW
EB2 = E_PAD // NW
EM = E_PAD // NS
CM = EM // K
N_ACC = 10240
NSTRIPE = N_ACC // NS

_MESH = plsc.VectorSubcoreMesh(core_axis_name="c", subcore_axis_name="s")


ZSTRIPE = 5120
NRP = NS * ZSTRIPE


@functools.partial(
    pl.kernel,
    out_type=[jax.ShapeDtypeStruct((E_PAD,), jnp.float32),
              jax.ShapeDtypeStruct((E_PAD,), jnp.int32)],
    mesh=_MESH,
    compiler_params=pltpu.CompilerParams(needs_layout_passes=False),
    scratch_types=[
        pltpu.VMEM((EM,), jnp.int32),
        pltpu.VMEM((EM,), jnp.int32),
        pltpu.VMEM((EM,), jnp.int32),
        pltpu.VMEM((EM,), jnp.float32),
        pltpu.VMEM((1, K), jnp.int32),
        pltpu.VMEM((1, K), jnp.int32),
        pltpu.VMEM((K,), jnp.int32),
        pltpu.VMEM((ZSTRIPE,), jnp.int32),
        pltpu.VMEM_SHARED((NRP,), jnp.int32),
    ],
)
def _edge_meta_kernel(src_hbm, dst_hbm, rel_hbm, zeros_hbm, w_hbm, g2_hbm,
                      bufa, relb, g2cb, wcb, keyb, cntc, onesb, stage, csh):
    c = lax.axis_index("c")
    s = lax.axis_index("s")
    base = s * EM
    pltpu.sync_copy(rel_hbm.at[pl.ds(base, EM)], relb)

    @pl.when(c == 1)
    def _():
        pltpu.sync_copy(src_hbm.at[pl.ds(base, EM)], bufa)

        def mk(i, _):
            sl = pl.ds(i * 16, 16)
            g2cb[sl] = (relb[sl] * N + bufa[sl]) * 2
            return 0
        lax.fori_loop(0, EM // 16, mk, 0)
        pltpu.sync_copy(g2cb, g2_hbm.at[pl.ds(base, EM)])

    @pl.when(c == 0)
    def _():
        pltpu.sync_copy(zeros_hbm, stage)
        pltpu.sync_copy(stage, csh.at[pl.ds(s * ZSTRIPE, ZSTRIPE)])
        pltpu.sync_copy(dst_hbm.at[pl.ds(base, EM)], bufa)
        for v in range(K // 16):
            onesb[pl.ds(v * 16, 16)] = jnp.ones((16,), jnp.int32)
        plsc.subcore_barrier()

        def mkkeys(j):
            for v in range(K // 16):
                esl = pl.ds(j * K + v * 16, 16)
                eid = base + j * K + v * 16 + lax.iota(jnp.int32, 16)
                key = jnp.where(eid < E, bufa[esl] * R + relb[esl],
                                N * R + (eid & 127))
                keyb[0, pl.ds(v * 16, 16)] = key

        def hchunk(j, _):
            mkkeys(j)
            pltpu.sync_copy(onesb, csh.at[keyb.at[0]], add=True)
            return 0
        lax.fori_loop(0, CM, hchunk, 0)
        plsc.subcore_barrier()

        def wchunk(j, _):
            mkkeys(j)
            pltpu.sync_copy(csh.at[keyb.at[0]], cntc.at[0])
            for v in range(K // 16):
                sl = pl.ds(v * 16, 16)
                cf = cntc[0, sl].astype(jnp.float32)
                eid = base + j * K + v * 16 + lax.iota(jnp.int32, 16)
                wcb[pl.ds(j * K + v * 16, 16)] = jnp.where(
                    eid < E, 1.0 / jnp.maximum(cf, 1.0), 0.0)
            return 0
        lax.fori_loop(0, CM, wchunk, 0)
        pltpu.sync_copy(wcb, w_hbm.at[pl.ds(base, EM)])


MG = 8


def _agg_body(oh, split, xw_hbm, g2_hbm, dst_hbm, w_hbm, zeros_hbm, agg_hbm,
              g2b, dstb, wb, idxb, rowsb, acc, gs0, gs1, ss0, ss1):
    c = lax.axis_index("c")
    s = lax.axis_index("s")
    pltpu.sync_copy(zeros_hbm, rowsb.at[0])
    for q in range(NSTRIPE // K):
        pltpu.async_copy(rowsb.at[0], acc.at[pl.ds(s * NSTRIPE + q * K, K)],
                         gs0)
    for q in range(NSTRIPE // K):
        pltpu.make_async_copy(rowsb.at[0],
                              acc.at[pl.ds(s * NSTRIPE + q * K, K)],
                              gs0).wait()
    plsc.subcore_barrier()
    ngroups = (CM if split else CM // 2) // MG
    gsems = (gs0, gs1)
    ssems = (ss0, ss1)

    def mkidx(u):
        b = u & 1
        for v in range(K // 16):
            sl = pl.ds(v * 16, 16)
            if split:
                idxb[b, sl] = g2b[u, sl] + c
            else:
                idxb[b, sl] = g2b[u, sl] >> 1

    def group(t, _):
        grow = (s * CM + t * MG) if split else (s * CM + c * (CM // 2) + t * MG)
        pltpu.sync_copy(g2_hbm.at[pl.ds(grow, MG)], g2b)
        pltpu.sync_copy(dst_hbm.at[pl.ds(grow, MG)], dstb)
        pltpu.sync_copy(w_hbm.at[pl.ds(grow, MG)], wb)
        mkidx(0)
        pltpu.async_copy(xw_hbm.at[idxb.at[0]], rowsb.at[0], gsems[0])
        for u in range(MG):
            b = u & 1
            b1 = (u + 1) & 1
            if u + 1 < MG:
                mkidx(u + 1)
                if u >= 1:
                    pltpu.make_async_copy(rowsb.at[b1],
                                          acc.at[dstb.at[u - 1]],
                                          ssems[b1]).wait()
                pltpu.async_copy(xw_hbm.at[idxb.at[b1]], rowsb.at[b1],
                                 gsems[b1])
            pltpu.make_async_copy(xw_hbm.at[idxb.at[b]], rowsb.at[b],
                                  gsems[b]).wait()

            def scale(g, _):
                wv = wb[u, pl.ds(g * 16, 16)]
                for l in range(16):
                    wk = wv[l]
                    k = g * 16 + l
                    for f in range(oh // 16):
                        fl = pl.ds(f * 16, 16)
                        rowsb[b, k, fl] = rowsb[b, k, fl] * wk
                return 0
            lax.fori_loop(0, K // 16, scale, 0)
            pltpu.async_copy(rowsb.at[b], acc.at[dstb.at[u]], ssems[b],
                             add=True)
        pltpu.make_async_copy(rowsb.at[0], acc.at[dstb.at[MG - 2]],
                              ssems[0]).wait()
        pltpu.make_async_copy(rowsb.at[1], acc.at[dstb.at[MG - 1]],
                              ssems[1]).wait()
        return 0
    lax.fori_loop(0, ngroups, group, 0)
    plsc.subcore_barrier()
    for q in range(NSTRIPE // K):
        b = q & 1
        rsl = pl.ds(s * NSTRIPE + q * K, K)
        if q >= 2:
            psl = pl.ds(s * NSTRIPE + (q - 2) * K, K)
            pltpu.make_async_copy(rowsb.at[b], agg_hbm.at[c, psl],
                                  ssems[b]).wait()
        pltpu.sync_copy(acc.at[rsl], rowsb.at[b])
        pltpu.async_copy(rowsb.at[b], agg_hbm.at[c, rsl], ssems[b])
    for q in range(NSTRIPE // K - 2, NSTRIPE // K):
        b = q & 1
        rsl = pl.ds(s * NSTRIPE + q * K, K)
        pltpu.make_async_copy(rowsb.at[b], agg_hbm.at[c, rsl],
                              ssems[b]).wait()


def _aggregate_sc(xw, g2_2d, dst_2d, w_2d, zeros, o_ch, split):
    oh = o_ch // 2 if split else o_ch
    xwf = xw.reshape(-1, oh)
    return pl.kernel(
        functools.partial(_agg_body, oh, split),
        out_type=jax.ShapeDtypeStruct((NC, N_ACC, oh), jnp.float32),
        mesh=_MESH,
        compiler_params=pltpu.CompilerParams(needs_layout_passes=False),
        scratch_types=[
            pltpu.VMEM((MG, K), jnp.int32),
            pltpu.VMEM((MG, K), jnp.int32),
            pltpu.VMEM((MG, K), jnp.float32),
            pltpu.VMEM((2, K), jnp.int32),
            pltpu.VMEM((2, K, oh), jnp.float32),
            pltpu.VMEM_SHARED((N_ACC, oh), jnp.float32),
            pltpu.SemaphoreType.DMA,
            pltpu.SemaphoreType.DMA,
            pltpu.SemaphoreType.DMA,
            pltpu.SemaphoreType.DMA,
        ],
    )(xwf, g2_2d, dst_2d, w_2d, zeros)


def kernel(x_mirna, x_gene, edge_index, edge_type, lin_m_w, lin_m_b,
           lin_g_w, lin_g_b, comp1, bases1, root1, bias1, comp2, bases2,
           root2, bias2):
    src = edge_index[0].astype(jnp.int32)
    dst = edge_index[1].astype(jnp.int32)
    rel = edge_type.astype(jnp.int32)

    npad = E_PAD - E
    spread = jnp.arange(npad, dtype=jnp.int32) % 997
    src_p = jnp.concatenate([src, spread])
    dst_p = jnp.concatenate([dst, spread])
    rel_p = jnp.concatenate([rel, jnp.zeros((npad,), jnp.int32)])

    zeros_h = jnp.zeros((ZSTRIPE,), jnp.int32)
    w, g2 = _edge_meta_kernel(src_p, dst_p, rel_p, zeros_h)
    w_2d = w.reshape(E_PAD // K, K)
    g2_2d = g2.reshape(E_PAD // K, K)
    dst_2d = dst_p.reshape(E_PAD // K, K)
    zeros1 = jnp.zeros((K, HID_CH // 2), jnp.float32)
    zeros2 = jnp.zeros((K, OUT_CH), jnp.float32)

    xw1, x = _proj_xw1(x_mirna, x_gene, lin_m_w, lin_g_w, lin_m_b, lin_g_b,
                       comp1, bases1)
    agg1 = _aggregate_sc(xw1, g2_2d, dst_2d, w_2d, zeros1, HID_CH, True)

    xw2, h = _combine_xw2(agg1, x, root1, bias1, comp2, bases2)
    agg2 = _aggregate_sc(xw2, g2_2d, dst_2d, w_2d, zeros2, OUT_CH, False)
    out = _combine(agg2, h, root2, bias2, HID_CH, OUT_CH, False)

    return (out[:N_MIRNA], out[N_MIRNA:])
